# baseline (speedup 1.0000x reference)
import functools

import jax
import jax.numpy as jnp
from jax import lax
from jax.experimental import pallas as pl
from jax.experimental.pallas import tpu as pltpu
from jax.experimental.pallas import tpu_sc as plsc

D = H = W = 192
N_ROWS = 3 * D * H
N_LANES = 16
NC = 2
NS = 16
NW = NC * NS
B_PER = 160
NPAD = NW * B_PER
CHUNKS = B_PER // N_LANES
WIN = 128
ZHI = W - WIN


def _make_sc_kernel():
    mesh = plsc.VectorSubcoreMesh(core_axis_name="c", subcore_axis_name="s")

    @functools.partial(
        pl.kernel,
        mesh=mesh,
        compiler_params=pltpu.CompilerParams(needs_layout_passes=False),
        out_type=jax.ShapeDtypeStruct((3 * NPAD,), jnp.float32),
        scratch_types=[
            pltpu.VMEM((3 * B_PER,), jnp.float32),
            pltpu.VMEM((3 * B_PER,), jnp.float32),
            pltpu.VMEM((3 * N_LANES,), jnp.float32),
            pltpu.VMEM((6 * N_LANES, WIN), jnp.float32),
            pltpu.VMEM((6 * N_LANES,), jnp.float32),
            pltpu.VMEM((3 * B_PER,), jnp.float32),
            pltpu.SemaphoreType.DMA,
        ],
    )
    def sc_kernel(mov_hbm, fix_hbm, sp_hbm, field_hbm, zhi_hbm, out_hbm,
                  mbuf, fbuf, spbuf, wins, zels, obuf, sem):
        wid = lax.axis_index("s") * NC + lax.axis_index("c")
        base = wid * B_PER

        for ch in range(3):
            pltpu.sync_copy(mov_hbm.at[pl.ds(ch * NPAD + base, B_PER)],
                            mbuf.at[pl.ds(ch * B_PER, B_PER)])
            pltpu.sync_copy(fix_hbm.at[pl.ds(ch * NPAD + base, B_PER)],
                            fbuf.at[pl.ds(ch * B_PER, B_PER)])
        pltpu.sync_copy(sp_hbm, spbuf)

        lanes = lax.iota(jnp.int32, N_LANES)

        for i in range(CHUNKS):
            mx = mbuf[pl.ds(0 * B_PER + i * N_LANES, N_LANES)]
            my = mbuf[pl.ds(1 * B_PER + i * N_LANES, N_LANES)]
            mz = mbuf[pl.ds(2 * B_PER + i * N_LANES, N_LANES)]
            fx = mx.astype(jnp.int32)
            fy = my.astype(jnp.int32)
            fz = mz.astype(jnp.int32)
            cx = jnp.where(mx > fx.astype(jnp.float32), fx + 1, fx)
            cy = jnp.where(my > fy.astype(jnp.float32), fy + 1, fy)
            cz = jnp.where(mz > fz.astype(jnp.float32), fz + 1, fz)

            copies = []
            for ch in range(3):
                for corner, (rx, ry, rz) in enumerate(
                        ((fx, fy, fz), (cx, cy, cz))):
                    row = (ch * D + rx) * H + ry
                    j = 2 * ch + corner
                    copies.append(pltpu.async_copy(
                        field_hbm.at[row, pl.ds(0, WIN)],
                        wins.at[pl.ds(j * N_LANES, N_LANES), :], sem))
                    zi = jnp.where(rz >= WIN, row * ZHI + rz - WIN, 0)
                    copies.append(pltpu.async_copy(
                        zhi_hbm.at[zi],
                        zels.at[pl.ds(j * N_LANES, N_LANES)], sem))
            for cp in copies:
                cp.wait()

            for ch in range(3):
                vals = []
                for corner, z in ((0, fz), (1, cz)):
                    j = 2 * ch + corner
                    in_a = z < WIN
                    za = jnp.where(in_a, z, 0)
                    ga = plsc.load_gather(wins, [lanes + j * N_LANES, za])
                    gb = zels[pl.ds(j * N_LANES, N_LANES)]
                    vals.append(jnp.where(in_a, ga, gb))
                m = mbuf[pl.ds(ch * B_PER + i * N_LANES, N_LANES)]
                fxl = fbuf[pl.ds(ch * B_PER + i * N_LANES, N_LANES)]
                sp = spbuf[pl.ds(ch * N_LANES, N_LANES)]
                obuf[pl.ds(ch * B_PER + i * N_LANES, N_LANES)] = (
                    (m + (vals[0] + vals[1]) * 0.5 - fxl) * sp)

        for ch in range(3):
            pltpu.sync_copy(obuf.at[pl.ds(ch * B_PER, B_PER)],
                            out_hbm.at[pl.ds(ch * NPAD + base, B_PER)])

    return sc_kernel


_SC_KERNEL = _make_sc_kernel()


def kernel(fixed_landmarks, moving_landmarks, displacement_field,
           fixed_spacing, moving_spacing):
    n = moving_landmarks.shape[0]
    mt = jnp.zeros((3, NPAD), jnp.float32).at[:, :n].set(
        moving_landmarks.T).reshape(3 * NPAD)
    ft = jnp.zeros((3, NPAD), jnp.float32).at[:, :n].set(
        fixed_landmarks.T).reshape(3 * NPAD)
    spb = jnp.broadcast_to(
        moving_spacing.astype(jnp.float32)[:, None],
        (3, N_LANES)).reshape(3 * N_LANES)
    field_rows = displacement_field.reshape(N_ROWS, W)
    zhi = displacement_field[:, :, :, :, WIN:].reshape(N_ROWS * ZHI)
    out_t = _SC_KERNEL(mt, ft, spb, field_rows, zhi)
    return out_t.reshape(3, NPAD)[:, :n].T


# speedup vs baseline: 1.0428x; 1.0428x over previous
import functools

import jax
import jax.numpy as jnp
from jax import lax
from jax.experimental import pallas as pl
from jax.experimental.pallas import tpu as pltpu
from jax.experimental.pallas import tpu_sc as plsc

D = H = W = 192
N_ROWS = 3 * D * H
N_LANES = 16
NC = 2
NS = 16
NW = NC * NS
B_PER = 160
NPAD = NW * B_PER
CHUNKS = B_PER // N_LANES
WIN = 128
ZHI = W - WIN


def _make_sc_kernel():
    mesh = plsc.VectorSubcoreMesh(core_axis_name="c", subcore_axis_name="s")

    @functools.partial(
        pl.kernel,
        mesh=mesh,
        compiler_params=pltpu.CompilerParams(needs_layout_passes=False),
        out_type=jax.ShapeDtypeStruct((3 * NPAD,), jnp.float32),
        scratch_types=[
            pltpu.VMEM((3 * B_PER,), jnp.float32),
            pltpu.VMEM((3 * B_PER,), jnp.float32),
            pltpu.VMEM((3 * N_LANES,), jnp.float32),
            pltpu.VMEM((6 * N_LANES, WIN), jnp.float32),
            pltpu.VMEM((6 * N_LANES, WIN), jnp.float32),
            pltpu.VMEM((6 * N_LANES,), jnp.float32),
            pltpu.VMEM((6 * N_LANES,), jnp.float32),
            pltpu.VMEM((3 * B_PER,), jnp.float32),
            pltpu.SemaphoreType.DMA,
            pltpu.SemaphoreType.DMA,
        ],
    )
    def sc_kernel(mov_hbm, fix_hbm, sp_hbm, field_hbm, zhi_hbm, out_hbm,
                  mbuf, fbuf, spbuf, wins0, wins1, zels0, zels1, obuf,
                  sem0, sem1):
        wid = lax.axis_index("s") * NC + lax.axis_index("c")
        base = wid * B_PER

        for ch in range(3):
            pltpu.sync_copy(mov_hbm.at[pl.ds(ch * NPAD + base, B_PER)],
                            mbuf.at[pl.ds(ch * B_PER, B_PER)])
            pltpu.sync_copy(fix_hbm.at[pl.ds(ch * NPAD + base, B_PER)],
                            fbuf.at[pl.ds(ch * B_PER, B_PER)])
        pltpu.sync_copy(sp_hbm, spbuf)

        lanes = lax.iota(jnp.int32, N_LANES)
        wins = (wins0, wins1)
        zels = (zels0, zels1)
        sems = (sem0, sem1)

        def corners(i):
            mx = mbuf[pl.ds(0 * B_PER + i * N_LANES, N_LANES)]
            my = mbuf[pl.ds(1 * B_PER + i * N_LANES, N_LANES)]
            mz = mbuf[pl.ds(2 * B_PER + i * N_LANES, N_LANES)]
            fx = mx.astype(jnp.int32)
            fy = my.astype(jnp.int32)
            fz = mz.astype(jnp.int32)
            cx = jnp.where(mx > fx.astype(jnp.float32), fx + 1, fx)
            cy = jnp.where(my > fy.astype(jnp.float32), fy + 1, fy)
            cz = jnp.where(mz > fz.astype(jnp.float32), fz + 1, fz)
            return ((fx, fy, fz), (cx, cy, cz))

        def fire(i):
            b = i % 2
            copies = []
            crn = corners(i)
            for ch in range(3):
                for corner in range(2):
                    rx, ry, rz = crn[corner]
                    row = (ch * D + rx) * H + ry
                    j = 2 * ch + corner
                    copies.append(pltpu.async_copy(
                        field_hbm.at[row, pl.ds(0, WIN)],
                        wins[b].at[pl.ds(j * N_LANES, N_LANES), :], sems[b]))
                    zi = jnp.where(rz >= WIN, row * WIN + rz - WIN, 0)
                    copies.append(pltpu.async_copy(
                        zhi_hbm.at[zi],
                        zels[b].at[pl.ds(j * N_LANES, N_LANES)], sems[b]))
            return copies

        def reduce(i, copies):
            b = i % 2
            for cp in copies:
                cp.wait()
            crn = corners(i)
            for ch in range(3):
                vals = []
                for corner in range(2):
                    z = crn[corner][2]
                    j = 2 * ch + corner
                    in_a = z < WIN
                    za = jnp.where(in_a, z, 0)
                    ga = plsc.load_gather(wins[b],
                                          [lanes + j * N_LANES, za])
                    gb = zels[b][pl.ds(j * N_LANES, N_LANES)]
                    vals.append(jnp.where(in_a, ga, gb))
                m = mbuf[pl.ds(ch * B_PER + i * N_LANES, N_LANES)]
                fxl = fbuf[pl.ds(ch * B_PER + i * N_LANES, N_LANES)]
                sp = spbuf[pl.ds(ch * N_LANES, N_LANES)]
                obuf[pl.ds(ch * B_PER + i * N_LANES, N_LANES)] = (
                    (m + (vals[0] + vals[1]) * 0.5 - fxl) * sp)

        inflight = fire(0)
        for i in range(CHUNKS):
            nxt = fire(i + 1) if i + 1 < CHUNKS else None
            reduce(i, inflight)
            inflight = nxt

        for ch in range(3):
            pltpu.sync_copy(obuf.at[pl.ds(ch * B_PER, B_PER)],
                            out_hbm.at[pl.ds(ch * NPAD + base, B_PER)])

    return sc_kernel


_SC_KERNEL = _make_sc_kernel()


def kernel(fixed_landmarks, moving_landmarks, displacement_field,
           fixed_spacing, moving_spacing):
    n = moving_landmarks.shape[0]
    mt = jnp.zeros((3, NPAD), jnp.float32).at[:, :n].set(
        moving_landmarks.T).reshape(3 * NPAD)
    ft = jnp.zeros((3, NPAD), jnp.float32).at[:, :n].set(
        fixed_landmarks.T).reshape(3 * NPAD)
    spb = jnp.broadcast_to(
        moving_spacing.astype(jnp.float32)[:, None],
        (3, N_LANES)).reshape(3 * N_LANES)
    field_rows = displacement_field.reshape(N_ROWS, W)
    zhi = jnp.pad(field_rows[:, WIN:], ((0, 0), (0, WIN - ZHI)))
    out_t = _SC_KERNEL(mt, ft, spb, field_rows, zhi.reshape(N_ROWS * WIN))
    return out_t.reshape(3, NPAD)[:, :n].T


# speedup vs baseline: 1.0433x; 1.0005x over previous
import functools

import jax
import jax.numpy as jnp
from jax import lax
from jax.experimental import pallas as pl
from jax.experimental.pallas import tpu as pltpu
from jax.experimental.pallas import tpu_sc as plsc

D = H = W = 192
N_ROWS = 3 * D * H
N_LANES = 16
NC = 2
NS = 16
NW = NC * NS
B_PER = 160
NPAD = NW * B_PER
CHUNKS = B_PER // N_LANES
WIN = 128
ZHI = W - WIN


def _make_sc_kernel():
    mesh = plsc.VectorSubcoreMesh(core_axis_name="c", subcore_axis_name="s")

    @functools.partial(
        pl.kernel,
        mesh=mesh,
        compiler_params=pltpu.CompilerParams(needs_layout_passes=False),
        out_type=jax.ShapeDtypeStruct((3 * NPAD,), jnp.float32),
        scratch_types=[
            pltpu.VMEM((3 * B_PER,), jnp.float32),
            pltpu.VMEM((3 * B_PER,), jnp.float32),
            pltpu.VMEM((3 * N_LANES,), jnp.float32),
            pltpu.VMEM((6 * N_LANES, WIN), jnp.float32),
            pltpu.VMEM((6 * N_LANES, WIN), jnp.float32),
            pltpu.VMEM((6 * N_LANES,), jnp.float32),
            pltpu.VMEM((6 * N_LANES,), jnp.float32),
            pltpu.VMEM((3 * B_PER,), jnp.float32),
            pltpu.SemaphoreType.DMA,
            pltpu.SemaphoreType.DMA,
        ],
    )
    def sc_kernel(mov_hbm, fix_hbm, sp_hbm, field_hbm, zhi_hbm, out_hbm,
                  mbuf, fbuf, spbuf, wins0, wins1, zels0, zels1, obuf,
                  sem0, sem1):
        wid = lax.axis_index("s") * NC + lax.axis_index("c")
        base = wid * B_PER

        for ch in range(3):
            pltpu.sync_copy(mov_hbm.at[pl.ds(ch * NPAD + base, B_PER)],
                            mbuf.at[pl.ds(ch * B_PER, B_PER)])
            pltpu.sync_copy(fix_hbm.at[pl.ds(ch * NPAD + base, B_PER)],
                            fbuf.at[pl.ds(ch * B_PER, B_PER)])
        pltpu.sync_copy(sp_hbm, spbuf)

        lanes = lax.iota(jnp.int32, N_LANES)
        wins = (wins0, wins1)
        zels = (zels0, zels1)
        sems = (sem0, sem1)

        def corners(i):
            mx = mbuf[pl.ds(0 * B_PER + i * N_LANES, N_LANES)]
            my = mbuf[pl.ds(1 * B_PER + i * N_LANES, N_LANES)]
            mz = mbuf[pl.ds(2 * B_PER + i * N_LANES, N_LANES)]
            fx = mx.astype(jnp.int32)
            fy = my.astype(jnp.int32)
            fz = mz.astype(jnp.int32)
            cx = jnp.where(mx > fx.astype(jnp.float32), fx + 1, fx)
            cy = jnp.where(my > fy.astype(jnp.float32), fy + 1, fy)
            cz = jnp.where(mz > fz.astype(jnp.float32), fz + 1, fz)
            return ((fx, fy, fz), (cx, cy, cz))

        def fire(i):
            b = i % 2
            copies = []
            crn = corners(i)
            for ch in range(3):
                for corner in range(2):
                    rx, ry, rz = crn[corner]
                    row = (ch * D + rx) * H + ry
                    j = 2 * ch + corner
                    copies.append(pltpu.async_copy(
                        field_hbm.at[row, pl.ds(0, WIN)],
                        wins[b].at[pl.ds(j * N_LANES, N_LANES), :], sems[b]))
                    zi = jnp.where(rz >= WIN, row * WIN + rz - WIN, 0)
                    copies.append(pltpu.async_copy(
                        zhi_hbm.at[zi],
                        zels[b].at[pl.ds(j * N_LANES, N_LANES)], sems[b]))
            return copies

        def reduce(i, copies):
            b = i % 2
            for cp in copies:
                cp.wait()
            crn = corners(i)
            for ch in range(3):
                vals = []
                for corner in range(2):
                    z = crn[corner][2]
                    j = 2 * ch + corner
                    in_a = z < WIN
                    za = jnp.where(in_a, z, 0)
                    ga = plsc.load_gather(wins[b],
                                          [lanes + j * N_LANES, za])
                    gb = zels[b][pl.ds(j * N_LANES, N_LANES)]
                    vals.append(jnp.where(in_a, ga, gb))
                m = mbuf[pl.ds(ch * B_PER + i * N_LANES, N_LANES)]
                fxl = fbuf[pl.ds(ch * B_PER + i * N_LANES, N_LANES)]
                sp = spbuf[pl.ds(ch * N_LANES, N_LANES)]
                obuf[pl.ds(ch * B_PER + i * N_LANES, N_LANES)] = (
                    (m + (vals[0] + vals[1]) * 0.5 - fxl) * sp)

        inflight = fire(0)
        for i in range(CHUNKS):
            nxt = fire(i + 1) if i + 1 < CHUNKS else None
            reduce(i, inflight)
            inflight = nxt

        for ch in range(3):
            pltpu.sync_copy(obuf.at[pl.ds(ch * B_PER, B_PER)],
                            out_hbm.at[pl.ds(ch * NPAD + base, B_PER)])

    return sc_kernel


_SC_KERNEL = _make_sc_kernel()


def kernel(fixed_landmarks, moving_landmarks, displacement_field,
           fixed_spacing, moving_spacing):
    n = moving_landmarks.shape[0]
    mt = jnp.zeros((3, NPAD), jnp.float32).at[:, :n].set(
        moving_landmarks.T).reshape(3 * NPAD)
    ft = jnp.zeros((3, NPAD), jnp.float32).at[:, :n].set(
        fixed_landmarks.T).reshape(3 * NPAD)
    spb = jnp.broadcast_to(
        moving_spacing.astype(jnp.float32)[:, None],
        (3, N_LANES)).reshape(3 * N_LANES)
    field_rows = displacement_field.reshape(N_ROWS, W)
    zhi = jnp.concatenate(
        [field_rows[:, WIN:], jnp.zeros((N_ROWS, WIN - ZHI), jnp.float32)],
        axis=1)
    out_t = _SC_KERNEL(mt, ft, spb, field_rows, zhi.reshape(N_ROWS * WIN))
    return out_t.reshape(3, NPAD)[:, :n].T
